# baseline (speedup 1.0000x reference)
import jax
import jax.numpy as jnp
from jax.experimental import pallas as pl
from jax.experimental.pallas import tpu as pltpu

FEATURE = 1024
QUEUE = 65536
BATCH = 4096
NCHUNK = 16
ROWS = FEATURE // NCHUNK
TCH = 8
TR = BATCH // TCH


def _bulk_copy(queue_ref, out_ref, sems, k):
    return pltpu.make_async_copy(
        queue_ref.at[pl.ds(k * ROWS, ROWS), pl.ds(BATCH, QUEUE - BATCH)],
        out_ref.at[pl.ds(k * ROWS, ROWS), pl.ds(BATCH, QUEUE - BATCH)],
        sems.at[k])


def _body(keys_ref, queue_ref, out_ref, kbuf, tbuf, sems, ksem, tsem):
    for k in range(NCHUNK):
        _bulk_copy(queue_ref, out_ref, sems, k).start()
    pltpu.make_async_copy(keys_ref, kbuf, ksem).start()
    pltpu.make_async_copy(keys_ref, kbuf, ksem).wait()
    for c in range(TCH):
        tbuf[:, c * TR:(c + 1) * TR] = kbuf[c * TR:(c + 1) * TR, :].T
    pltpu.make_async_copy(tbuf, out_ref.at[:, pl.ds(0, BATCH)], tsem).start()
    pltpu.make_async_copy(tbuf, out_ref.at[:, pl.ds(0, BATCH)], tsem).wait()
    for k in range(NCHUNK):
        _bulk_copy(queue_ref, out_ref, sems, k).wait()


def kernel(keys, queue):
    return pl.pallas_call(
        _body,
        in_specs=[
            pl.BlockSpec(memory_space=pltpu.MemorySpace.HBM),
            pl.BlockSpec(memory_space=pltpu.MemorySpace.HBM),
        ],
        out_specs=pl.BlockSpec(memory_space=pltpu.MemorySpace.HBM),
        out_shape=jax.ShapeDtypeStruct((FEATURE, QUEUE), jnp.float32),
        scratch_shapes=[
            pltpu.VMEM((BATCH, FEATURE), jnp.float32),
            pltpu.VMEM((FEATURE, BATCH), jnp.float32),
            pltpu.SemaphoreType.DMA((NCHUNK,)),
            pltpu.SemaphoreType.DMA,
            pltpu.SemaphoreType.DMA,
        ],
    )(keys, queue)


# speedup vs baseline: 43.7591x; 43.7591x over previous
import jax
import jax.numpy as jnp
from jax.experimental import pallas as pl

FEATURE = 1024
QUEUE = 65536
BATCH = 4096
BLOCK_R = 512
BLOCK_C = 4096
GRID_R = FEATURE // BLOCK_R
GRID_C = QUEUE // BLOCK_C


def _body(keys_ref, queue_ref, out_ref):
    j = pl.program_id(1)

    @pl.when(j == 0)
    def _():
        out_ref[...] = keys_ref[...].T

    @pl.when(j > 0)
    def _():
        out_ref[...] = queue_ref[...]


def kernel(keys, queue):
    return pl.pallas_call(
        _body,
        grid=(GRID_R, GRID_C),
        in_specs=[
            pl.BlockSpec((BATCH, BLOCK_R), lambda i, j: (0, i)),
            pl.BlockSpec((BLOCK_R, BLOCK_C),
                         lambda i, j: (i, jnp.maximum(j, 1))),
        ],
        out_specs=pl.BlockSpec((BLOCK_R, BLOCK_C), lambda i, j: (i, j)),
        out_shape=jax.ShapeDtypeStruct((FEATURE, QUEUE), jnp.float32),
    )(keys, queue)


# speedup vs baseline: 45.4886x; 1.0395x over previous
import jax
import jax.numpy as jnp
from jax.experimental import pallas as pl
from jax.experimental.pallas import tpu as pltpu

FEATURE = 1024
QUEUE = 65536
BATCH = 4096
C = 1024
K = 4
NB = (QUEUE - BATCH) // C
T = NB // K
TCH = 4
TR = BATCH // TCH


def _bulk_in(queue_ref, sbuf, isems, t, k):
    c = t * K + k
    return pltpu.make_async_copy(
        queue_ref.at[:, pl.ds(BATCH + c * C, C)],
        sbuf.at[k, t % 2], isems.at[k, t % 2])


def _bulk_out(out_ref, sbuf, osems, t, k):
    c = t * K + k
    return pltpu.make_async_copy(
        sbuf.at[k, t % 2],
        out_ref.at[:, pl.ds(BATCH + c * C, C)], osems.at[k, t % 2])


def _t_out(out_ref, tbuf, tsems, r):
    return pltpu.make_async_copy(
        tbuf.at[r % 2],
        out_ref.at[:, pl.ds(r * TR, TR)], tsems.at[r % 2])


def _body(keys_ref, queue_ref, out_ref, kbuf, tbuf, sbuf,
          ksem, tsems, isems, osems):
    kfetch = pltpu.make_async_copy(keys_ref, kbuf, ksem)
    kfetch.start()
    for k in range(K):
        _bulk_in(queue_ref, sbuf, isems, 0, k).start()
    kfetch.wait()
    for t in range(T):
        for k in range(K):
            _bulk_in(queue_ref, sbuf, isems, t, k).wait()
            if t >= 1:
                _bulk_out(out_ref, sbuf, osems, t - 1, k).wait()
            if t + 1 < T:
                _bulk_in(queue_ref, sbuf, isems, t + 1, k).start()
            _bulk_out(out_ref, sbuf, osems, t, k).start()
        if t < TCH:
            if t >= 2:
                _t_out(out_ref, tbuf, tsems, t - 2).wait()
            tbuf[t % 2] = kbuf[t * TR:(t + 1) * TR, :].T
            _t_out(out_ref, tbuf, tsems, t).start()
    for k in range(K):
        _bulk_out(out_ref, sbuf, osems, T - 1, k).wait()
    for r in (TCH - 2, TCH - 1):
        _t_out(out_ref, tbuf, tsems, r).wait()


def kernel(keys, queue):
    return pl.pallas_call(
        _body,
        in_specs=[
            pl.BlockSpec(memory_space=pltpu.MemorySpace.HBM),
            pl.BlockSpec(memory_space=pltpu.MemorySpace.HBM),
        ],
        out_specs=pl.BlockSpec(memory_space=pltpu.MemorySpace.HBM),
        out_shape=jax.ShapeDtypeStruct((FEATURE, QUEUE), jnp.float32),
        scratch_shapes=[
            pltpu.VMEM((BATCH, FEATURE), jnp.float32),
            pltpu.VMEM((2, FEATURE, TR), jnp.float32),
            pltpu.VMEM((K, 2, FEATURE, C), jnp.float32),
            pltpu.SemaphoreType.DMA,
            pltpu.SemaphoreType.DMA((2,)),
            pltpu.SemaphoreType.DMA((K, 2)),
            pltpu.SemaphoreType.DMA((K, 2)),
        ],
    )(keys, queue)
